# baseline (speedup 1.0000x reference)
import functools

import jax
import jax.numpy as jnp
from jax import lax
from jax.experimental import pallas as pl
from jax.experimental.pallas import tpu as pltpu
from jax.experimental.pallas import tpu_sc as plsc

_VOCAB = 1000000
_HIDDEN = 64
_BATCH = 4096
_SEQ = 200

_info = plsc.get_sparse_core_info()
_NC, _NS = _info.num_cores, _info.num_subcores
_NW = _NC * _NS
_BPW = _BATCH // _NW
_IPW = _BPW * _SEQ
_S0 = 104
_S1 = _SEQ - _S0
_NSLOT = 4
_LOOKAHEAD = 2


def _sc_body(idx_hbm, emb_hbm, pos_hbm, out_hbm,
             idx_v, pos_v, rows, gsem, wsem, psem):
    wid = lax.axis_index("s") * _NC + lax.axis_index("c")
    pltpu.sync_copy(idx_hbm.at[pl.ds(wid * _IPW, _IPW)], idx_v)
    pltpu.sync_copy(pos_hbm.at[pl.ds(0, _SEQ)], pos_v)

    def gathers(j, slot):
        pltpu.make_async_copy(emb_hbm.at[idx_v.at[pl.ds(j * _SEQ, _S0)]],
                              rows.at[slot, pl.ds(0, _S0)],
                              gsem.at[slot]).start()
        pltpu.make_async_copy(emb_hbm.at[idx_v.at[pl.ds(j * _SEQ + _S0, _S1)]],
                              rows.at[slot, pl.ds(_S0, _S1)],
                              gsem.at[slot]).start()

    def wait_gathers(j, slot):
        pltpu.make_async_copy(emb_hbm.at[idx_v.at[pl.ds(j * _SEQ, _S0)]],
                              rows.at[slot, pl.ds(0, _S0)],
                              gsem.at[slot]).wait()
        pltpu.make_async_copy(emb_hbm.at[idx_v.at[pl.ds(j * _SEQ + _S0, _S1)]],
                              rows.at[slot, pl.ds(_S0, _S1)],
                              gsem.at[slot]).wait()

    def writes_start(j, slot):
        b = wid * _BPW + j
        pltpu.make_async_copy(
            pos_v, out_hbm.at[b, :, pl.ds(0, _HIDDEN)],
            psem.at[slot]).start()
        pltpu.make_async_copy(
            rows.at[slot],
            out_hbm.at[b, :, pl.ds(_HIDDEN, _HIDDEN)],
            wsem.at[slot]).start()

    def writes_wait(j, slot):
        b = wid * _BPW + j
        pltpu.make_async_copy(
            pos_v, out_hbm.at[b, :, pl.ds(0, _HIDDEN)],
            psem.at[slot]).wait()
        pltpu.make_async_copy(
            rows.at[slot],
            out_hbm.at[b, :, pl.ds(_HIDDEN, _HIDDEN)],
            wsem.at[slot]).wait()

    for j in range(_LOOKAHEAD):
        gathers(j, j % _NSLOT)

    def body(j, carry):
        slot = j % _NSLOT
        wait_gathers(j, slot)
        writes_start(j, slot)

        @pl.when(j + _LOOKAHEAD < _BPW)
        def _():
            ns = (j + _LOOKAHEAD) % _NSLOT

            @pl.when(j + _LOOKAHEAD >= _NSLOT)
            def _():
                writes_wait(j + _LOOKAHEAD - _NSLOT, ns)

            gathers(j + _LOOKAHEAD, ns)

        return carry

    lax.fori_loop(0, _BPW, body, 0)

    for j in range(_BPW - _NSLOT, _BPW):
        writes_wait(j, j % _NSLOT)


_TC = 512
_TGRID = (_VOCAB + _TC - 1) // _TC
_TROWS = _TGRID * _TC // 2


def _transpose_body(in_ref, out_ref):
    x = in_ref[...]
    out_ref[...] = jnp.concatenate(
        [x[:, : _TC // 2].T, x[:, _TC // 2:].T], axis=1)


def _linearize_table(embedding):
    emb_t = embedding.T
    out = pl.pallas_call(
        _transpose_body,
        grid=(_TGRID,),
        in_specs=[pl.BlockSpec((_HIDDEN, _TC), lambda i: (0, i))],
        out_specs=pl.BlockSpec((_TC // 2, 2 * _HIDDEN), lambda i: (i, 0)),
        out_shape=jax.ShapeDtypeStruct((_TROWS, 128), jnp.float32),
    )(emb_t)
    return out.reshape(2 * _TROWS, _HIDDEN)


def _permute_indices(idx):
    t = idx.reshape(-1)
    return (t & ~jnp.int32(511)) | ((t & 255) << 1) | ((t >> 8) & 1)


@functools.partial(jax.jit, static_argnums=())
def _run(idx, embedding, position_embedding):
    mesh = plsc.VectorSubcoreMesh(core_axis_name="c", subcore_axis_name="s")
    kern = pl.kernel(
        _sc_body,
        mesh=mesh,
        compiler_params=pltpu.CompilerParams(use_tc_tiling_on_sc=False),
        out_type=jax.ShapeDtypeStruct((_BATCH, _SEQ, 2 * _HIDDEN),
                                      jnp.float32),
        scratch_types=[
            pltpu.VMEM((_IPW,), jnp.int32),
            pltpu.VMEM((_SEQ, _HIDDEN), jnp.float32),
            pltpu.VMEM((_NSLOT, _SEQ, _HIDDEN), jnp.float32),
            pltpu.SemaphoreType.DMA((_NSLOT,)),
            pltpu.SemaphoreType.DMA((_NSLOT,)),
            pltpu.SemaphoreType.DMA((_NSLOT,)),
        ],
    )
    return kern(_permute_indices(idx), _linearize_table(embedding),
                position_embedding)


def kernel(inputs, embedding, position_embedding):
    return _run(inputs, embedding, position_embedding)


# speedup vs baseline: 1.8717x; 1.8717x over previous
import functools

import jax
import jax.numpy as jnp
from jax import lax
from jax.experimental import pallas as pl
from jax.experimental.pallas import tpu as pltpu
from jax.experimental.pallas import tpu_sc as plsc

_VOCAB = 1000000
_HIDDEN = 64
_BATCH = 4096
_SEQ = 200

_info = plsc.get_sparse_core_info()
_NC, _NS = _info.num_cores, _info.num_subcores
_NW = _NC * _NS
_BPW = _BATCH // _NW
_IPW = _BPW * _SEQ
_S0 = 104
_S1 = _SEQ - _S0
_NSLOT = 4
_LOOKAHEAD = 2


def _sc_body(idx_hbm, emb_hbm, pos_hbm, out_hbm,
             idx_v, pos_v, rows, gsem, wsem, psem):
    wid = lax.axis_index("s") * _NC + lax.axis_index("c")
    pltpu.sync_copy(idx_hbm.at[pl.ds(wid * _IPW, _IPW)], idx_v)
    pltpu.sync_copy(pos_hbm.at[pl.ds(0, _SEQ)], pos_v)

    def gathers(j, slot):
        pltpu.make_async_copy(emb_hbm.at[idx_v.at[pl.ds(j * _SEQ, _S0)]],
                              rows.at[slot, pl.ds(0, _S0)],
                              gsem.at[slot]).start()
        pltpu.make_async_copy(emb_hbm.at[idx_v.at[pl.ds(j * _SEQ + _S0, _S1)]],
                              rows.at[slot, pl.ds(_S0, _S1)],
                              gsem.at[slot]).start()

    def wait_gathers(j, slot):
        pltpu.make_async_copy(emb_hbm.at[idx_v.at[pl.ds(j * _SEQ, _S0)]],
                              rows.at[slot, pl.ds(0, _S0)],
                              gsem.at[slot]).wait()
        pltpu.make_async_copy(emb_hbm.at[idx_v.at[pl.ds(j * _SEQ + _S0, _S1)]],
                              rows.at[slot, pl.ds(_S0, _S1)],
                              gsem.at[slot]).wait()

    def writes_start(j, slot):
        b = wid * _BPW + j
        pltpu.make_async_copy(
            pos_v, out_hbm.at[b, :, pl.ds(0, _HIDDEN)],
            psem.at[slot]).start()
        pltpu.make_async_copy(
            rows.at[slot],
            out_hbm.at[b, :, pl.ds(_HIDDEN, _HIDDEN)],
            wsem.at[slot]).start()

    def writes_wait(j, slot):
        b = wid * _BPW + j
        pltpu.make_async_copy(
            pos_v, out_hbm.at[b, :, pl.ds(0, _HIDDEN)],
            psem.at[slot]).wait()
        pltpu.make_async_copy(
            rows.at[slot],
            out_hbm.at[b, :, pl.ds(_HIDDEN, _HIDDEN)],
            wsem.at[slot]).wait()

    for j in range(_LOOKAHEAD):
        gathers(j, j % _NSLOT)

    def body(j, carry):
        slot = j % _NSLOT
        wait_gathers(j, slot)
        writes_start(j, slot)

        @pl.when(j + _LOOKAHEAD < _BPW)
        def _():
            ns = (j + _LOOKAHEAD) % _NSLOT

            @pl.when(j + _LOOKAHEAD >= _NSLOT)
            def _():
                writes_wait(j + _LOOKAHEAD - _NSLOT, ns)

            gathers(j + _LOOKAHEAD, ns)

        return carry

    lax.fori_loop(0, _BPW, body, 0)

    for j in range(_BPW - _NSLOT, _BPW):
        writes_wait(j, j % _NSLOT)


def _linearize_table(embedding):
    wide = jnp.concatenate(
        [embedding, jnp.zeros((_VOCAB, _HIDDEN), jnp.float32)], axis=1)
    return wide.reshape(2 * _VOCAB, _HIDDEN)


def _permute_indices(idx):
    return idx.reshape(-1) * 2


@functools.partial(jax.jit, static_argnums=())
def _run(idx, embedding, position_embedding):
    mesh = plsc.VectorSubcoreMesh(core_axis_name="c", subcore_axis_name="s")
    kern = pl.kernel(
        _sc_body,
        mesh=mesh,
        compiler_params=pltpu.CompilerParams(use_tc_tiling_on_sc=False),
        out_type=jax.ShapeDtypeStruct((_BATCH, _SEQ, 2 * _HIDDEN),
                                      jnp.float32),
        scratch_types=[
            pltpu.VMEM((_IPW,), jnp.int32),
            pltpu.VMEM((_SEQ, _HIDDEN), jnp.float32),
            pltpu.VMEM((_NSLOT, _SEQ, _HIDDEN), jnp.float32),
            pltpu.SemaphoreType.DMA((_NSLOT,)),
            pltpu.SemaphoreType.DMA((_NSLOT,)),
            pltpu.SemaphoreType.DMA((_NSLOT,)),
        ],
    )
    return kern(_permute_indices(idx), _linearize_table(embedding),
                position_embedding)


def kernel(inputs, embedding, position_embedding):
    return _run(inputs, embedding, position_embedding)


# speedup vs baseline: 2.0630x; 1.1022x over previous
import functools

import jax
import jax.numpy as jnp
from jax import lax
from jax.experimental import pallas as pl
from jax.experimental.pallas import tpu as pltpu
from jax.experimental.pallas import tpu_sc as plsc

_VOCAB = 1000000
_HIDDEN = 64
_BATCH = 4096
_SEQ = 200

_info = plsc.get_sparse_core_info()
_NC, _NS = _info.num_cores, _info.num_subcores
_NW = _NC * _NS
_BPW = _BATCH // _NW
_IPW = _BPW * _SEQ
_S0 = 104
_S1 = _SEQ - _S0
_NSLOT = 4
_LOOKAHEAD = 2


def _sc_body(idx_hbm, emb_hbm, pos_hbm, out_hbm,
             idx_v, pos_v, rows, gsem, wsem, psem):
    wid = lax.axis_index("s") * _NC + lax.axis_index("c")
    pltpu.sync_copy(idx_hbm.at[pl.ds(wid * _IPW, _IPW)], idx_v)
    pltpu.sync_copy(pos_hbm.at[pl.ds(0, _SEQ)], pos_v)

    def gathers(j, slot):
        pltpu.make_async_copy(emb_hbm.at[idx_v.at[pl.ds(j * _SEQ, _S0)]],
                              rows.at[slot, pl.ds(0, _S0)],
                              gsem.at[slot]).start()
        pltpu.make_async_copy(emb_hbm.at[idx_v.at[pl.ds(j * _SEQ + _S0, _S1)]],
                              rows.at[slot, pl.ds(_S0, _S1)],
                              gsem.at[slot]).start()

    def wait_gathers(j, slot):
        pltpu.make_async_copy(emb_hbm.at[idx_v.at[pl.ds(j * _SEQ, _S0)]],
                              rows.at[slot, pl.ds(0, _S0)],
                              gsem.at[slot]).wait()
        pltpu.make_async_copy(emb_hbm.at[idx_v.at[pl.ds(j * _SEQ + _S0, _S1)]],
                              rows.at[slot, pl.ds(_S0, _S1)],
                              gsem.at[slot]).wait()

    def writes_start(j, slot):
        b = wid * _BPW + j
        pltpu.make_async_copy(
            pos_v, out_hbm.at[b, :, pl.ds(0, _HIDDEN)],
            psem.at[slot]).start()
        pltpu.make_async_copy(
            rows.at[slot],
            out_hbm.at[b, :, pl.ds(_HIDDEN, _HIDDEN)],
            wsem.at[slot]).start()

    def writes_wait(j, slot):
        b = wid * _BPW + j
        pltpu.make_async_copy(
            pos_v, out_hbm.at[b, :, pl.ds(0, _HIDDEN)],
            psem.at[slot]).wait()
        pltpu.make_async_copy(
            rows.at[slot],
            out_hbm.at[b, :, pl.ds(_HIDDEN, _HIDDEN)],
            wsem.at[slot]).wait()

    for j in range(_LOOKAHEAD):
        gathers(j, j % _NSLOT)

    def body(j, carry):
        slot = j % _NSLOT
        wait_gathers(j, slot)
        writes_start(j, slot)

        @pl.when(j + _LOOKAHEAD < _BPW)
        def _():
            ns = (j + _LOOKAHEAD) % _NSLOT

            @pl.when(j + _LOOKAHEAD >= _NSLOT)
            def _():
                writes_wait(j + _LOOKAHEAD - _NSLOT, ns)

            gathers(j + _LOOKAHEAD, ns)

        return carry

    lax.fori_loop(0, _BPW, body, 0)

    for j in range(_BPW - _NSLOT, _BPW):
        writes_wait(j, j % _NSLOT)


_TC = 2048
_TGRID = (_VOCAB + _TC - 1) // _TC
_TROWS = _TGRID * _TC // 2


def _transpose_body(in_ref, out_ref):
    x = in_ref[...]
    out_ref[...] = jnp.concatenate(
        [x[:, : _TC // 2].T, x[:, _TC // 2:].T], axis=1)


def _linearize_table(embedding):
    emb_t = embedding.T
    out = pl.pallas_call(
        _transpose_body,
        grid=(_TGRID,),
        in_specs=[pl.BlockSpec((_HIDDEN, _TC), lambda i: (0, i))],
        out_specs=pl.BlockSpec((_TC // 2, 2 * _HIDDEN), lambda i: (i, 0)),
        out_shape=jax.ShapeDtypeStruct((_TROWS, 128), jnp.float32),
    )(emb_t)
    return out.reshape(2 * _TROWS, _HIDDEN)


def _permute_indices(idx):
    t = idx.reshape(-1)
    return (t & ~jnp.int32(_TC - 1)) | ((t & (_TC // 2 - 1)) << 1) \
        | ((t >> 10) & 1)


@functools.partial(jax.jit, static_argnums=())
def _run(idx, embedding, position_embedding):
    mesh = plsc.VectorSubcoreMesh(core_axis_name="c", subcore_axis_name="s")
    kern = pl.kernel(
        _sc_body,
        mesh=mesh,
        compiler_params=pltpu.CompilerParams(use_tc_tiling_on_sc=False),
        out_type=jax.ShapeDtypeStruct((_BATCH, _SEQ, 2 * _HIDDEN),
                                      jnp.float32),
        scratch_types=[
            pltpu.VMEM((_IPW,), jnp.int32),
            pltpu.VMEM((_SEQ, _HIDDEN), jnp.float32),
            pltpu.VMEM((_NSLOT, _SEQ, _HIDDEN), jnp.float32),
            pltpu.SemaphoreType.DMA((_NSLOT,)),
            pltpu.SemaphoreType.DMA((_NSLOT,)),
            pltpu.SemaphoreType.DMA((_NSLOT,)),
        ],
    )
    return kern(_permute_indices(idx), _linearize_table(embedding),
                position_embedding)


def kernel(inputs, embedding, position_embedding):
    return _run(inputs, embedding, position_embedding)
